# baseline (speedup 1.0000x reference)
import jax
import jax.numpy as jnp
from jax.experimental import pallas as pl
from jax.experimental.pallas import tpu as pltpu
from jax.experimental.pallas import tpu_sc as plsc

BATCH = 4096
HIST = 200
DIM = 64
NUM_IDX = BATCH * HIST
WINDOW = 128


def kernel(x, emb_weight):
    idx = x.reshape(1, NUM_IDX).astype(jnp.int32)
    mesh = plsc.VectorSubcoreMesh(
        core_axis_name="core", subcore_axis_name="subcore"
    )

    @pl.kernel(
        out_type=jax.ShapeDtypeStruct((NUM_IDX, DIM), emb_weight.dtype),
        mesh=mesh,
        compiler_params=pltpu.CompilerParams(use_tc_tiling_on_sc=False),
    )
    def k(table_hbm, idx_hbm, out_hbm):
        def body(idx_vmem, out_vmem):
            pltpu.sync_copy(table_hbm.at[idx_vmem.at[0]], out_vmem)

        pltpu.emit_pipeline(
            body,
            grid=(NUM_IDX // WINDOW,),
            in_specs=[
                pl.BlockSpec((1, WINDOW), index_map=lambda i: (0, i))
            ],
            out_specs=[
                pl.BlockSpec((WINDOW, DIM), index_map=lambda i: (i, 0))
            ],
            core_axis_name=("core", "subcore"),
            dimension_semantics=(pltpu.PARALLEL,),
        )(idx_hbm, out_hbm)

    out = k(emb_weight, idx)
    return out.reshape(BATCH, HIST, DIM)


# speedup vs baseline: 1.0724x; 1.0724x over previous
import jax
import jax.numpy as jnp
from jax.experimental import pallas as pl
from jax.experimental.pallas import tpu as pltpu
from jax.experimental.pallas import tpu_sc as plsc

BATCH = 4096
HIST = 200
DIM = 64
NUM_IDX = BATCH * HIST
WINDOW = 512


def kernel(x, emb_weight):
    idx = x.reshape(1, NUM_IDX).astype(jnp.int32)
    mesh = plsc.VectorSubcoreMesh(
        core_axis_name="core", subcore_axis_name="subcore"
    )

    @pl.kernel(
        out_type=jax.ShapeDtypeStruct((NUM_IDX, DIM), emb_weight.dtype),
        mesh=mesh,
        compiler_params=pltpu.CompilerParams(use_tc_tiling_on_sc=False),
    )
    def k(table_hbm, idx_hbm, out_hbm):
        def body(idx_vmem, out_vmem):
            pltpu.sync_copy(table_hbm.at[idx_vmem.at[0]], out_vmem)

        pltpu.emit_pipeline(
            body,
            grid=(NUM_IDX // WINDOW,),
            in_specs=[
                pl.BlockSpec((1, WINDOW), index_map=lambda i: (0, i))
            ],
            out_specs=[
                pl.BlockSpec((WINDOW, DIM), index_map=lambda i: (i, 0))
            ],
            core_axis_name=("core", "subcore"),
            dimension_semantics=(pltpu.PARALLEL,),
        )(idx_hbm, out_hbm)

    out = k(emb_weight, idx)
    return out.reshape(BATCH, HIST, DIM)
